# baseline (speedup 1.0000x reference)
import functools

import jax
import jax.numpy as jnp
from jax import lax
from jax.experimental import pallas as pl
from jax.experimental.pallas import tpu as pltpu
from jax.experimental.pallas import tpu_sc as plsc

_V = 1000000
_D = 64
_B = 16384
_L = 50
_NC = 2
_NS = 16
_NW = _NC * _NS
_BW = _B // _NW
_NCOL = _BW // 128
_NGRP = _NCOL * _L

_mesh = plsc.VectorSubcoreMesh(core_axis_name="c", subcore_axis_name="s")


@functools.partial(
    pl.kernel,
    out_type=jax.ShapeDtypeStruct((_L, _D, _B), jnp.float32),
    scratch_types=[
        pltpu.VMEM((64, _BW), jnp.int32),
        pltpu.VMEM((2, 128, 128), jnp.float32),
        pltpu.VMEM((2, _D, 128), jnp.float32),
        pltpu.SemaphoreType.DMA((2,)),
        pltpu.SemaphoreType.DMA((2,)),
    ],
    mesh=_mesh,
    compiler_params=pltpu.CompilerParams(
        use_tc_tiling_on_sc=True, needs_layout_passes=False),
)
def _emb_tiled(xt_hbm, tab_hbm, out_hbm, slab, gbuf, obuf, gsem, ssem):
    wid = lax.axis_index("s") * _NC + lax.axis_index("c")
    b0 = wid * _BW
    pltpu.sync_copy(xt_hbm.at[:, pl.ds(b0, _BW)], slab)
    iota16 = lax.iota(jnp.int32, 16)

    def lc(q):
        c = q // _L
        return q - _L * c, c

    def gather_desc(q, d):
        l, c = lc(q)
        return pltpu.make_async_copy(
            tab_hbm.at[slab.at[l, pl.ds(128 * c, 128)]],
            gbuf.at[d], gsem.at[d])

    def store_desc(q, d):
        l, c = lc(q)
        return pltpu.make_async_copy(
            obuf.at[d],
            out_hbm.at[l, :, pl.ds(b0 + 128 * c, 128)],
            ssem.at[d])

    def transpose_into_obuf(d):
        def per_trow(t, carry):
            for k in range(8):
                rows = iota16 + 16 * k
                for s in range(8):
                    e = 8 * t + s
                    cols = jnp.full((16,), e, jnp.int32)
                    vals = plsc.load_gather(gbuf.at[d], [rows, cols])
                    obuf[d, e, pl.ds(16 * k, 16)] = vals
            return carry
        lax.fori_loop(0, 8, per_trow, 0)

    gather_desc(0, 0).start()

    def outer(q2, carry):
        for d in range(2):
            q = 2 * q2 + d

            @pl.when(q + 1 < _NGRP)
            def _():
                gather_desc(q + 1, 1 - d).start()

            gather_desc(q, d).wait()

            @pl.when(q >= 2)
            def _():
                store_desc(q, d).wait()

            transpose_into_obuf(d)
            store_desc(q, d).start()
        return carry

    lax.fori_loop(0, _NGRP // 2, outer, 0)

    for q in (_NGRP - 2, _NGRP - 1):
        store_desc(q, q % 2).wait()


def kernel(x, table):
    xt = jnp.pad(x.astype(jnp.int32).T, ((0, 64 - _L), (0, 0)))
    tp = jnp.pad(table, ((0, 0), (0, 128 - _D)))
    out3 = _emb_tiled(xt, tp)
    return jnp.transpose(out3, (2, 0, 1))


# speedup vs baseline: 1.2150x; 1.2150x over previous
import functools

import jax
import jax.numpy as jnp
from jax import lax
from jax.experimental import pallas as pl
from jax.experimental.pallas import tpu as pltpu
from jax.experimental.pallas import tpu_sc as plsc

_V = 1000000
_D = 64
_B = 16384
_L = 50
_NC = 2
_NS = 16
_NW = _NC * _NS
_BW = _B // _NW
_NCOL = _BW // 128
_NGRP = _NCOL * _L

_mesh = plsc.VectorSubcoreMesh(core_axis_name="c", subcore_axis_name="s")


@functools.partial(
    pl.kernel,
    out_type=jax.ShapeDtypeStruct((_L, _D, _B), jnp.float32),
    scratch_types=[
        pltpu.VMEM((64, _BW), jnp.int32),
        pltpu.VMEM((2, 128, 128), jnp.float32),
        pltpu.VMEM((2, _D, 128), jnp.float32),
        pltpu.SemaphoreType.DMA((2,)),
        pltpu.SemaphoreType.DMA((2,)),
    ],
    mesh=_mesh,
    compiler_params=pltpu.CompilerParams(
        use_tc_tiling_on_sc=True, needs_layout_passes=False),
)
def _emb_tiled(xt_hbm, tab_hbm, out_hbm, slab, gbuf, obuf, gsem, ssem):
    wid = lax.axis_index("s") * _NC + lax.axis_index("c")
    b0 = wid * _BW
    pltpu.sync_copy(xt_hbm.at[:, pl.ds(b0, _BW)], slab)
    iota16 = lax.iota(jnp.int32, 16)

    def lc(q):
        c = q // _L
        return q - _L * c, c

    def gather_desc(q, d):
        l, c = lc(q)
        return pltpu.make_async_copy(
            tab_hbm.at[slab.at[l, pl.ds(128 * c, 128)]],
            gbuf.at[d], gsem.at[d])

    def store_desc(q, d):
        l, c = lc(q)
        return pltpu.make_async_copy(
            obuf.at[d],
            out_hbm.at[l, :, pl.ds(b0 + 128 * c, 128)],
            ssem.at[d])

    def transpose_into_obuf(d):
        @plsc.parallel_loop(0, _D * 8, 1, unroll=8)
        def _(i):
            e = i // 8
            k = i - 8 * e
            rows = iota16 + 16 * k
            cols = jnp.full((16,), e, jnp.int32)
            vals = plsc.load_gather(gbuf.at[d], [rows, cols])
            obuf[d, e, pl.ds(16 * k, 16)] = vals

    gather_desc(0, 0).start()

    def outer(q2, carry):
        for d in range(2):
            q = 2 * q2 + d

            @pl.when(q + 1 < _NGRP)
            def _():
                gather_desc(q + 1, 1 - d).start()

            gather_desc(q, d).wait()

            @pl.when(q >= 2)
            def _():
                store_desc(q, d).wait()

            transpose_into_obuf(d)
            store_desc(q, d).start()
        return carry

    lax.fori_loop(0, _NGRP // 2, outer, 0)

    for q in (_NGRP - 2, _NGRP - 1):
        store_desc(q, q % 2).wait()


def kernel(x, table):
    xt = jnp.pad(x.astype(jnp.int32).T, ((0, 64 - _L), (0, 0)))
    tp = jnp.pad(table, ((0, 0), (0, 128 - _D)))
    out3 = _emb_tiled(xt, tp)
    return jnp.transpose(out3, (2, 0, 1))


# speedup vs baseline: 1.4367x; 1.1825x over previous
import functools

import jax
import jax.numpy as jnp
from jax import lax
from jax.experimental import pallas as pl
from jax.experimental.pallas import tpu as pltpu
from jax.experimental.pallas import tpu_sc as plsc

_V = 1000000
_D = 64
_B = 16384
_L = 50
_NC = 2
_NS = 16
_NW = _NC * _NS
_BW = _B // _NW
_NCOL = _BW // 128
_NGRP = _NCOL * _L

_mesh = plsc.VectorSubcoreMesh(core_axis_name="c", subcore_axis_name="s")


@functools.partial(
    pl.kernel,
    out_type=jax.ShapeDtypeStruct((_L, _D, _B), jnp.float32),
    scratch_types=[
        pltpu.VMEM((64, _BW), jnp.int32),
        pltpu.VMEM((2, 128, 128), jnp.float32),
        pltpu.VMEM((2, _D, 128), jnp.float32),
        pltpu.SemaphoreType.DMA((2,)),
        pltpu.SemaphoreType.DMA((2,)),
    ],
    mesh=_mesh,
    compiler_params=pltpu.CompilerParams(
        use_tc_tiling_on_sc=True, needs_layout_passes=False),
)
def _emb_tiled(xt_hbm, tab_hbm, out_hbm, slab, gbuf, obuf, gsem, ssem):
    wid = lax.axis_index("s") * _NC + lax.axis_index("c")
    b0 = wid * _BW
    pltpu.sync_copy(xt_hbm.at[:, pl.ds(b0, _BW)], slab)
    iota16 = lax.iota(jnp.int32, 16)

    def lc(q):
        c = q // _L
        return q - _L * c, c

    def gather_desc(q, d):
        l, c = lc(q)
        return pltpu.make_async_copy(
            tab_hbm.at[slab.at[l, pl.ds(128 * c, 128)]],
            gbuf.at[d], gsem.at[d])

    def store_desc(q, d):
        l, c = lc(q)
        return pltpu.make_async_copy(
            obuf.at[d],
            out_hbm.at[l, :, pl.ds(b0 + 128 * c, 128)],
            ssem.at[d])

    def transpose_into_obuf(d):
        @plsc.parallel_loop(0, _D * 8, 1, unroll=16)
        def _(i):
            k = i // _D
            e = i - _D * k
            rows = iota16 + 16 * k
            cols = jnp.full((16,), e, jnp.int32)
            vals = plsc.load_gather(gbuf.at[d], [rows, cols])
            obuf[d, e, pl.ds(16 * k, 16)] = vals

    gather_desc(0, 0).start()

    def outer(q2, carry):
        for d in range(2):
            q = 2 * q2 + d

            @pl.when(q + 1 < _NGRP)
            def _():
                gather_desc(q + 1, 1 - d).start()

            gather_desc(q, d).wait()

            @pl.when(q >= 2)
            def _():
                store_desc(q, d).wait()

            transpose_into_obuf(d)
            store_desc(q, d).start()
        return carry

    lax.fori_loop(0, _NGRP // 2, outer, 0)

    for q in (_NGRP - 2, _NGRP - 1):
        store_desc(q, q % 2).wait()


def kernel(x, table):
    xt = jnp.pad(x.astype(jnp.int32).T, ((0, 64 - _L), (0, 0)))
    tp = jnp.pad(table, ((0, 0), (0, 128 - _D)))
    out3 = _emb_tiled(xt, tp)
    return jnp.transpose(out3, (2, 0, 1))
